# baseline (speedup 1.0000x reference)
import functools

import jax
import jax.numpy as jnp
from jax import lax
from jax.experimental import pallas as pl
from jax.experimental.pallas import tpu as pltpu
from jax.experimental.pallas import tpu_sc as plsc

N_NODES = 10000
N_EDGES = 320000
D_FEAT = 128
D_AGG = D_FEAT + 3
N_CORES = 2
N_SUBCORES = 16
NW = N_CORES * N_SUBCORES
E_PER_W = N_EDGES // NW
CHUNK = 80
N_CHUNKS = E_PER_W // CHUNK
GROUPS = CHUNK // 16
PROWS = CHUNK // 8
NBUF = 5
GATHER_LEAD = 2
OUT_LAG = 3

_mesh = plsc.VectorSubcoreMesh(core_axis_name="c", subcore_axis_name="s")


@functools.partial(
    pl.kernel,
    mesh=_mesh,
    out_type=[
        jax.ShapeDtypeStruct((N_EDGES, D_FEAT), jnp.float32),
        jax.ShapeDtypeStruct((N_EDGES // 8, 128), jnp.float32),
    ],
    scratch_types=[
        pltpu.VMEM((N_NODES * 3,), jnp.float32),
        pltpu.VMEM((E_PER_W,), jnp.int32),
        pltpu.VMEM((E_PER_W,), jnp.int32),
        pltpu.VMEM((NBUF * CHUNK, D_FEAT), jnp.float32),
        pltpu.VMEM((NBUF * PROWS, 128), jnp.float32),
    ] + [pltpu.SemaphoreType.DMA] * (2 * NBUF),
    compiler_params=pltpu.CompilerParams(
        needs_layout_passes=False, use_tc_tiling_on_sc=False),
)
def _edge_kernel(feat_hbm, pos_hbm, src_hbm, dst_hbm, feat_out_hbm,
                 pos_out_hbm, pos_v, sidx_v, didx_v, feat_v, pack_v, *sems):
    gsem = sems[:NBUF]
    osem = sems[NBUF:]
    wid = lax.axis_index("s") * N_CORES + lax.axis_index("c")
    ebase = wid * E_PER_W
    pbase = wid * (E_PER_W // 8)
    pltpu.sync_copy(pos_hbm, pos_v)
    pltpu.sync_copy(src_hbm.at[pl.ds(ebase, E_PER_W)], sidx_v)
    pltpu.sync_copy(dst_hbm.at[pl.ds(ebase, E_PER_W)], didx_v)

    iota = lax.iota(jnp.int32, 16)

    def gather_start(c, b):
        pltpu.async_copy(
            feat_hbm.at[sidx_v.at[pl.ds(c * CHUNK, CHUNK)]],
            feat_v.at[pl.ds(b * CHUNK, CHUNK)], gsem[b])

    def gather_wait(c, b):
        pltpu.make_async_copy(
            feat_hbm.at[sidx_v.at[pl.ds(c * CHUNK, CHUNK)]],
            feat_v.at[pl.ds(b * CHUNK, CHUNK)], gsem[b]).wait()

    def out_start(c, b):
        pltpu.async_copy(feat_v.at[pl.ds(b * CHUNK, CHUNK)],
                         feat_out_hbm.at[pl.ds(ebase + c * CHUNK, CHUNK)],
                         osem[b])
        pltpu.async_copy(pack_v.at[pl.ds(b * PROWS, PROWS)],
                         pos_out_hbm.at[pl.ds(pbase + c * PROWS, PROWS)],
                         osem[b])

    def out_wait(c, b):
        pltpu.make_async_copy(
            feat_v.at[pl.ds(b * CHUNK, CHUNK)],
            feat_out_hbm.at[pl.ds(ebase + c * CHUNK, CHUNK)],
            osem[b]).wait()
        pltpu.make_async_copy(
            pack_v.at[pl.ds(b * PROWS, PROWS)],
            pos_out_hbm.at[pl.ds(pbase + c * PROWS, PROWS)],
            osem[b]).wait()

    def compute(c, b):
        off = c * CHUNK
        for k in range(GROUPS):
            el = k * 16 + iota
            row = b * PROWS + el // 8
            lane = (el % 8) * 16
            sidx = sidx_v[pl.ds(off + k * 16, 16)]
            didx = didx_v[pl.ds(off + k * 16, 16)]
            s3 = sidx * 3
            d3 = didx * 3
            ps = [plsc.load_gather(pos_v, [s3 + cc]) for cc in range(3)]
            pd = [plsc.load_gather(pos_v, [d3 + cc]) for cc in range(3)]
            for cc in range(3):
                plsc.store_scatter(pack_v, [row, lane + cc], ps[cc] - pd[cc])
                plsc.store_scatter(pack_v, [row, lane + (3 + cc)], ps[cc])
                plsc.store_scatter(pack_v, [row, lane + (6 + cc)], pd[cc])

    def slot(c, b, do_outwait, do_gstart):
        b2 = (b + GATHER_LEAD) % NBUF
        if do_outwait:
            out_wait(c - OUT_LAG, b2)
        if do_gstart:
            gather_start(c + GATHER_LEAD, b2)
        gather_wait(c, b)
        compute(c, b)
        out_start(c, b)

    for c in range(GATHER_LEAD):
        gather_start(c, c)
    for c in range(NBUF):
        slot(c, c, do_outwait=(c >= OUT_LAG), do_gstart=True)

    @pl.loop(NBUF, N_CHUNKS - NBUF, step=NBUF)
    def _steady(g):
        for b in range(NBUF):
            slot(g + b, b, do_outwait=True, do_gstart=True)

    for c in range(N_CHUNKS - NBUF, N_CHUNKS):
        slot(c, c % NBUF, do_outwait=True,
             do_gstart=(c + GATHER_LEAD < N_CHUNKS))
    for c in range(N_CHUNKS - OUT_LAG, N_CHUNKS):
        out_wait(c, c % NBUF)


def kernel(pos, feat, edge_index):
    src = edge_index[0]
    dst = edge_index[1]
    feat_out, pos_out = _edge_kernel(feat, pos.reshape(-1), src, dst)
    pk = pos_out.reshape(N_EDGES // 8, 8, 16)
    rel = pk[:, :, 0:3].reshape(N_EDGES, 3)
    ps = pk[:, :, 3:6].reshape(N_EDGES, 3)
    pd = pk[:, :, 6:9].reshape(N_EDGES, 3)
    agg = jnp.concatenate([rel, feat_out], axis=1)
    geo = jnp.concatenate([ps, pd], axis=1)
    return (agg, geo)


# speedup vs baseline: 1.1954x; 1.1954x over previous
import functools

import jax
import jax.numpy as jnp
from jax import lax
from jax.experimental import pallas as pl
from jax.experimental.pallas import tpu as pltpu
from jax.experimental.pallas import tpu_sc as plsc

N_NODES = 10000
N_EDGES = 320000
D_FEAT = 128
D_AGG = D_FEAT + 3
N_CORES = 2
N_SUBCORES = 16
NW = N_CORES * N_SUBCORES
E_PER_W = N_EDGES // NW
CHUNK = 80
N_CHUNKS = E_PER_W // CHUNK
GROUPS = CHUNK // 16
PROWS = CHUNK // 8
NBUF = 5
GATHER_LEAD = 2
OUT_LAG = 3

_mesh = plsc.VectorSubcoreMesh(core_axis_name="c", subcore_axis_name="s")


@functools.partial(
    pl.kernel,
    mesh=_mesh,
    out_type=[
        jax.ShapeDtypeStruct((N_EDGES, D_FEAT), jnp.float32),
        jax.ShapeDtypeStruct((N_EDGES // 8, 128), jnp.float32),
    ],
    scratch_types=[
        pltpu.VMEM((N_NODES * 3,), jnp.float32),
        pltpu.VMEM((E_PER_W,), jnp.int32),
        pltpu.VMEM((E_PER_W,), jnp.int32),
        pltpu.VMEM((NBUF * CHUNK, D_FEAT), jnp.float32),
        pltpu.VMEM((NBUF * PROWS, 128), jnp.float32),
    ] + [pltpu.SemaphoreType.DMA] * (2 * NBUF),
    compiler_params=pltpu.CompilerParams(
        needs_layout_passes=False, use_tc_tiling_on_sc=False),
)
def _edge_kernel(feat_hbm, pos_hbm, src_hbm, dst_hbm, feat_out_hbm,
                 pos_out_hbm, pos_v, sidx_v, didx_v, feat_v, pack_v, *sems):
    gsem = sems[:NBUF]
    osem = sems[NBUF:]
    wid = lax.axis_index("s") * N_CORES + lax.axis_index("c")
    ebase = wid * E_PER_W
    pbase = wid * (E_PER_W // 8)
    pltpu.sync_copy(pos_hbm, pos_v)
    pltpu.sync_copy(src_hbm.at[pl.ds(ebase, E_PER_W)], sidx_v)
    pltpu.sync_copy(dst_hbm.at[pl.ds(ebase, E_PER_W)], didx_v)

    iota = lax.iota(jnp.int32, 16)

    def gather_start(c, b):
        pltpu.async_copy(
            feat_hbm.at[sidx_v.at[pl.ds(c * CHUNK, CHUNK)]],
            feat_v.at[pl.ds(b * CHUNK, CHUNK)], gsem[b])

    def gather_wait(c, b):
        pltpu.make_async_copy(
            feat_hbm.at[sidx_v.at[pl.ds(c * CHUNK, CHUNK)]],
            feat_v.at[pl.ds(b * CHUNK, CHUNK)], gsem[b]).wait()

    def out_start(c, b):
        pltpu.async_copy(feat_v.at[pl.ds(b * CHUNK, CHUNK)],
                         feat_out_hbm.at[pl.ds(ebase + c * CHUNK, CHUNK)],
                         osem[b])
        pltpu.async_copy(pack_v.at[pl.ds(b * PROWS, PROWS)],
                         pos_out_hbm.at[pl.ds(pbase + c * PROWS, PROWS)],
                         osem[b])

    def out_wait(c, b):
        pltpu.make_async_copy(
            feat_v.at[pl.ds(b * CHUNK, CHUNK)],
            feat_out_hbm.at[pl.ds(ebase + c * CHUNK, CHUNK)],
            osem[b]).wait()
        pltpu.make_async_copy(
            pack_v.at[pl.ds(b * PROWS, PROWS)],
            pos_out_hbm.at[pl.ds(pbase + c * PROWS, PROWS)],
            osem[b]).wait()

    def compute(c, b):
        off = c * CHUNK
        for k in range(GROUPS):
            el = k * 16 + iota
            row = b * PROWS + el // 8
            lane = (el % 8) * 16
            sidx = sidx_v[pl.ds(off + k * 16, 16)]
            didx = didx_v[pl.ds(off + k * 16, 16)]
            s3 = sidx * 3
            d3 = didx * 3
            ps = [plsc.load_gather(pos_v, [s3 + cc]) for cc in range(3)]
            pd = [plsc.load_gather(pos_v, [d3 + cc]) for cc in range(3)]
            for cc in range(3):
                plsc.store_scatter(pack_v, [row, lane + cc], ps[cc] - pd[cc])
                plsc.store_scatter(pack_v, [row, lane + (3 + cc)], ps[cc])
                plsc.store_scatter(pack_v, [row, lane + (6 + cc)], pd[cc])

    def slot(c, b, do_outwait, do_gstart):
        b2 = (b + GATHER_LEAD) % NBUF
        if do_outwait:
            out_wait(c - OUT_LAG, b2)
        if do_gstart:
            gather_start(c + GATHER_LEAD, b2)
        gather_wait(c, b)
        compute(c, b)
        out_start(c, b)

    for c in range(GATHER_LEAD):
        gather_start(c, c)
    for c in range(NBUF):
        slot(c, c, do_outwait=(c >= OUT_LAG), do_gstart=True)

    @pl.loop(NBUF, N_CHUNKS - NBUF, step=NBUF)
    def _steady(g):
        for b in range(NBUF):
            slot(g + b, b, do_outwait=True, do_gstart=True)

    for c in range(N_CHUNKS - NBUF, N_CHUNKS):
        slot(c, c % NBUF, do_outwait=True,
             do_gstart=(c + GATHER_LEAD < N_CHUNKS))
    for c in range(N_CHUNKS - OUT_LAG, N_CHUNKS):
        out_wait(c, c % NBUF)


def kernel(pos, feat, edge_index):
    src = edge_index[0]
    dst = edge_index[1]
    feat_out, pos_out = _edge_kernel(feat, pos.reshape(-1), src, dst)
    pk = pos_out.reshape(N_EDGES // 8, 8, 16)
    rel = pk[:, :, 0:3].reshape(N_EDGES, 3)
    geo = pk[:, :, 3:9].reshape(N_EDGES, 6)
    agg = jnp.concatenate([rel, feat_out], axis=1)
    return (agg, geo)
